# baseline (speedup 1.0000x reference)
import jax
import jax.numpy as jnp
from jax import lax
from jax.experimental import pallas as pl

N_EXPERTS = 64
TOP_K = 2
BLOCK_N = 2048
NSTREAM = 2


def _router_kernel(*refs):
    x_refs = refs[:NSTREAM]
    w_refs = refs[NSTREAM : 2 * NSTREAM]
    b_ref = refs[2 * NSTREAM]
    gates_ref, idx_ref = refs[2 * NSTREAM + 1 :]

    logits = b_ref[...]
    for s in range(NSTREAM):
        logits = logits + jnp.dot(
            x_refs[s][...], w_refs[s][...], preferred_element_type=jnp.float32
        )

    e = lax.broadcasted_iota(jnp.int32, logits.shape, 1)

    m1 = jnp.max(logits, axis=1, keepdims=True)
    i1 = jnp.min(jnp.where(logits == m1, e, N_EXPERTS), axis=1, keepdims=True)

    masked = jnp.where(e == i1, -jnp.inf, logits)
    m2 = jnp.max(masked, axis=1, keepdims=True)
    i2 = jnp.min(jnp.where(masked == m2, e, N_EXPERTS), axis=1, keepdims=True)

    e2 = jnp.exp(m2 - m1)
    denom = 1.0 + e2
    p1 = 1.0 / denom
    p2 = e2 / denom

    gates_ref[...] = jnp.where(e == i1, p1, 0.0) + jnp.where(e == i2, p2, 0.0)
    idx_ref[...] = jnp.concatenate([i1, i2], axis=1)


@jax.jit
def kernel(x, W, b):
    n, d = x.shape
    ds = d // NSTREAM
    grid = (n // BLOCK_N,)

    in_specs = [
        pl.BlockSpec((BLOCK_N, ds), lambda i, s=s: (i, s)) for s in range(NSTREAM)
    ]
    in_specs += [
        pl.BlockSpec((ds, N_EXPERTS), lambda i, s=s: (s, 0)) for s in range(NSTREAM)
    ]
    in_specs.append(pl.BlockSpec((1, N_EXPERTS), lambda i: (0, 0)))
    out_specs = [
        pl.BlockSpec((BLOCK_N, N_EXPERTS), lambda i: (i, 0)),
        pl.BlockSpec((BLOCK_N, TOP_K), lambda i: (i, 0)),
    ]
    out_shape = [
        jax.ShapeDtypeStruct((n, N_EXPERTS), jnp.float32),
        jax.ShapeDtypeStruct((n, TOP_K), jnp.int32),
    ]

    gates, idx = pl.pallas_call(
        _router_kernel,
        grid=grid,
        in_specs=in_specs,
        out_specs=out_specs,
        out_shape=out_shape,
    )(*([x] * NSTREAM), *([W] * NSTREAM), b.reshape(1, N_EXPERTS))
    return (gates, idx)
